# baseline (speedup 1.0000x reference)
import functools

import jax
import jax.numpy as jnp
from jax import lax
from jax.experimental import pallas as pl
from jax.experimental.pallas import tpu as pltpu
from jax.experimental.pallas import tpu_sc as plsc

D_MODEL = 768
D_VOCAB = 100000
NC = 2
NS = 16
NW = NC * NS
T = 8192
TPW = T // NW
GT = 16
NG = TPW // GT
IDX_PER_G = GT * D_MODEL
NSTREAM = 8
SLEN = IDX_PER_G // NSTREAM


def _embed_body(w_hbm, x_hbm, out_hbm, xv, idx0, idx1, g0, g1,
                gs0, gs1, os0, os1):
    wid = lax.axis_index("s") * NC + lax.axis_index("c")
    tok0 = wid * TPW
    pltpu.sync_copy(x_hbm.at[pl.ds(tok0, TPW)], xv)

    iota = lax.iota(jnp.int32, 16)
    pos_base = iota * D_MODEL
    idxs = (idx0, idx1)
    gbufs = (g0, g1)
    gsems = (gs0, gs1)
    osems = (os0, os1)
    gather_cp = [None, None]
    out_cp = [None, None]

    def build(b, g):
        v_vec = xv[pl.ds(g * GT, GT)]
        ref = idxs[b]

        def body(d, c):
            pos = pos_base + d
            val = v_vec + d * D_VOCAB
            plsc.store_scatter(ref, [pos], val)
            return c

        lax.fori_loop(0, D_MODEL, body, 0)

    def fire_gathers(b):
        cps = []
        for j in range(NSTREAM):
            cps.append(pltpu.async_copy(
                w_hbm.at[idxs[b].at[pl.ds(j * SLEN, SLEN)]],
                gbufs[b].at[pl.ds(j * SLEN, SLEN)],
                gsems[b]))
        return cps

    def fire_out(b, g):
        w0 = (tok0 + g * GT) * D_MODEL
        return pltpu.async_copy(gbufs[b], out_hbm.at[pl.ds(w0, IDX_PER_G)],
                                osems[b])

    for g in range(NG):
        b = g & 1
        build(b, g)
        if out_cp[b] is not None:
            out_cp[b].wait()
        gather_cp[b] = fire_gathers(b)
        if g >= 1:
            pb = (g - 1) & 1
            for cp in gather_cp[pb]:
                cp.wait()
            out_cp[pb] = fire_out(pb, g - 1)

    b = (NG - 1) & 1
    for cp in gather_cp[b]:
        cp.wait()
    out_cp[b] = fire_out(b, NG - 1)
    out_cp[0].wait()
    out_cp[1].wait()


@functools.partial(
    pl.kernel,
    out_type=jax.ShapeDtypeStruct((T * D_MODEL,), jnp.float32),
    mesh=plsc.VectorSubcoreMesh(core_axis_name="c", subcore_axis_name="s"),
    compiler_params=pltpu.CompilerParams(needs_layout_passes=False),
    scratch_types=[
        pltpu.VMEM((TPW,), jnp.int32),
        pltpu.VMEM((IDX_PER_G,), jnp.int32),
        pltpu.VMEM((IDX_PER_G,), jnp.int32),
        pltpu.VMEM((IDX_PER_G,), jnp.float32),
        pltpu.VMEM((IDX_PER_G,), jnp.float32),
        pltpu.SemaphoreType.DMA,
        pltpu.SemaphoreType.DMA,
        pltpu.SemaphoreType.DMA,
        pltpu.SemaphoreType.DMA,
    ],
)
def _embed_call(w_hbm, x_hbm, out_hbm, xv, idx0, idx1, g0, g1,
                gs0, gs1, os0, os1):
    _embed_body(w_hbm, x_hbm, out_hbm, xv, idx0, idx1, g0, g1,
                gs0, gs1, os0, os1)


def kernel(x, W_E):
    b, s = x.shape
    xf = x.reshape(-1).astype(jnp.int32)
    wf = W_E.reshape(-1)
    out = _embed_call(wf, xf)
    return out.reshape(b, s, D_MODEL)


# speedup vs baseline: 1.7677x; 1.7677x over previous
import functools

import jax
import jax.numpy as jnp
from jax import lax
from jax.experimental import pallas as pl
from jax.experimental.pallas import tpu as pltpu
from jax.experimental.pallas import tpu_sc as plsc

D_MODEL = 768
D_VOCAB = 100000
NC = 2
NS = 16
NW = NC * NS
T = 8192
NBLK = D_VOCAB // 128
TAIL0 = NBLK * 128
TAILB0 = D_VOCAB - 128
QD = 192
NQ = D_MODEL // QD
XCH = T // 16
RING = 8
SENTINEL = 0x7FFFFFFF


def _embed_body(w_hbm, tail_hbm, x_hbm, out_hbm, xtile, wlist, blist,
                qb0, qb1, stag, qs0, qs1, osem):
    wid = lax.axis_index("s") * NC + lax.axis_index("c")
    pltpu.sync_copy(x_hbm, xtile)

    iota = lax.iota(jnp.int32, 16)
    c0 = (wid * NBLK) >> 5
    c1 = ((wid + 1) * NBLK) >> 5

    def compact(dst, off, e, m):
        cs = plsc.cumsum(m.astype(jnp.int32))
        pos = jnp.where(m, off + cs - 1, 0)
        plsc.store_scatter(dst, [pos], e, mask=m)
        return off + cs[15]

    def scan_body(k, off):
        v = xtile[pl.ds(k * 16, 16)]
        e = (v << 13) | (iota + k * 16)
        blk = v >> 7
        return compact(wlist, off, e, (blk >= c0) & (blk < c1))

    wcount = lax.fori_loop(0, XCH, scan_body, 0)
    plsc.store_scatter(wlist, [wcount + iota],
                       jnp.full((16,), jnp.int32(SENTINEL)))
    plsc.store_scatter(wlist, [wcount + 16 + iota],
                       jnp.full((16,), jnp.int32(SENTINEL)))
    nwch = (wcount + 31) >> 4

    bufs = (qb0, qb1)
    sems = (qs0, qs1)

    def fire_q(cblk, q, buf, sem):
        src = w_hbm.at[pl.ds((q % NQ) * QD, QD),
                       pl.ds(pl.multiple_of(cblk * 128, 128), 128)]
        return pltpu.async_copy(src, buf, sem)

    def drain_q(buf, sem):
        pltpu.make_async_copy(
            w_hbm.at[pl.ds(0, QD), pl.ds(0, 128)], buf, sem).wait()

    def drain_out(slot, nwords):
        pltpu.make_async_copy(
            stag.at[pl.ds(slot * D_MODEL, nwords)],
            out_hbm.at[pl.ds(0, nwords)],
            osem.at[slot]).wait()

    fire_q(c0, 0, qb0, qs0)
    fire_q(c0, 1, qb1, qs1)

    def extract_rows(buf, nm, rows_per_tok, colv_fn, out_off_fn):
        def tok(i, drain):
            ch = blist[pl.ds((i >> 4) * 16, 16)]
            e = jnp.take_along_axis(
                ch, jnp.full((16,), i & 15, jnp.int32), axis=0)
            colv = colv_fn(e)
            t = e[0] & 8191
            slot = i & (RING - 1)
            sbase = slot * D_MODEL
            if drain:
                drain_out(slot, rows_per_tok)
            for j in range(rows_per_tok // 16):
                val = plsc.load_gather(buf, [iota + 16 * j, colv])
                stag[pl.ds(sbase + 16 * j, 16)] = val
            pltpu.async_copy(
                stag.at[pl.ds(pl.multiple_of(sbase, 8), rows_per_tok)],
                out_hbm.at[pl.ds(
                    pl.multiple_of(out_off_fn(t), 8), rows_per_tok)],
                osem.at[slot])
            return 0

        lax.fori_loop(0, jnp.minimum(nm, RING),
                      lambda i, c: tok(i, False), 0)
        lax.fori_loop(RING, jnp.maximum(nm, RING),
                      lambda i, c: tok(i, True), 0)
        lax.fori_loop(0, jnp.minimum(nm, RING),
                      lambda s, c: (drain_out(s, rows_per_tok), c)[1], 0)

    def block_body(cb, carry):
        def bl_body(k, off):
            ch = wlist[pl.ds(k * 16, 16)]
            return compact(blist, off, ch, (ch >> 20) == cb)

        nm = lax.fori_loop(0, nwch, bl_body, 0)

        for q in range(NQ):
            b = q & 1
            drain_q(bufs[b], sems[b])
            extract_rows(
                bufs[b], nm, QD,
                lambda e: (e >> 13) & 127,
                lambda t, q=q: t * D_MODEL + q * QD)
            if q < 2:
                fire_q(cb, q + 2, bufs[b], sems[b])
            else:
                fire_q(jnp.minimum(cb + 1, NBLK - 1), q - 2,
                       bufs[b], sems[b])
        return carry

    lax.fori_loop(c0, c1, block_body, 0)
    drain_q(qb0, qs0)
    drain_q(qb1, qs1)

    @pl.when(wid == NW - 1)
    def _tail():
        def tscan(k, off):
            v = xtile[pl.ds(k * 16, 16)]
            e = (v << 13) | (iota + k * 16)
            return compact(blist, off, e, v >= TAIL0)

        nm = lax.fori_loop(0, XCH, tscan, 0)

        for q in range(NQ):
            pltpu.sync_copy(tail_hbm.at[pl.ds(q * QD, QD)], qb0)
            extract_rows(
                qb0, nm, QD,
                lambda e: (e >> 13) - TAILB0,
                lambda t, q=q: t * D_MODEL + q * QD)


@functools.partial(
    pl.kernel,
    out_type=jax.ShapeDtypeStruct((T * D_MODEL,), jnp.float32),
    mesh=plsc.VectorSubcoreMesh(core_axis_name="c", subcore_axis_name="s"),
    compiler_params=pltpu.CompilerParams(needs_layout_passes=False),
    scratch_types=[
        pltpu.VMEM((T,), jnp.int32),
        pltpu.VMEM((T + 32,), jnp.int32),
        pltpu.VMEM((T + 32,), jnp.int32),
        pltpu.VMEM((QD, 128), jnp.float32),
        pltpu.VMEM((QD, 128), jnp.float32),
        pltpu.VMEM((RING * D_MODEL,), jnp.float32),
        pltpu.SemaphoreType.DMA,
        pltpu.SemaphoreType.DMA,
        pltpu.SemaphoreType.DMA((RING,)),
    ],
)
def _embed_call(w_hbm, tail_hbm, x_hbm, out_hbm, xtile, wlist, blist,
                qb0, qb1, stag, qs0, qs1, osem):
    _embed_body(w_hbm, tail_hbm, x_hbm, out_hbm, xtile, wlist, blist,
                qb0, qb1, stag, qs0, qs1, osem)


def kernel(x, W_E):
    b, s = x.shape
    xf = x.reshape(-1).astype(jnp.int32)
    tail = lax.slice(W_E, (0, TAILB0), (D_MODEL, D_VOCAB))
    out = _embed_call(W_E, tail, xf)
    return out.reshape(b, s, D_MODEL)


# speedup vs baseline: 1.8107x; 1.0243x over previous
import functools

import jax
import jax.numpy as jnp
from jax import lax
from jax.experimental import pallas as pl
from jax.experimental.pallas import tpu as pltpu
from jax.experimental.pallas import tpu_sc as plsc

D_MODEL = 768
D_VOCAB = 100000
NC = 2
NS = 16
NW = NC * NS
T = 8192
NBLK = D_VOCAB // 128
TAIL0 = NBLK * 128
TAILB0 = D_VOCAB - 128
QD = 384
NQ = D_MODEL // QD
XCH = T // 16
RING = 4
SENTINEL = 0x7FFFFFFF


def _embed_body(w_hbm, tail_hbm, x_hbm, out_hbm, xtile, wlist, blist,
                qb0, qb1, stag, qs0, qs1, osem):
    wid = lax.axis_index("s") * NC + lax.axis_index("c")
    pltpu.sync_copy(x_hbm, xtile)

    iota = lax.iota(jnp.int32, 16)
    c0 = (wid * NBLK) >> 5
    c1 = ((wid + 1) * NBLK) >> 5

    def compact(dst, off, e, m):
        cs = plsc.cumsum(m.astype(jnp.int32))
        pos = jnp.where(m, off + cs - 1, 0)
        plsc.store_scatter(dst, [pos], e, mask=m)
        return off + cs[15]

    def scan_body(k, off):
        v = xtile[pl.ds(k * 16, 16)]
        e = (v << 13) | (iota + k * 16)
        blk = v >> 7
        return compact(wlist, off, e, (blk >= c0) & (blk < c1))

    wcount = lax.fori_loop(0, XCH, scan_body, 0)
    plsc.store_scatter(wlist, [wcount + iota],
                       jnp.full((16,), jnp.int32(SENTINEL)))
    plsc.store_scatter(wlist, [wcount + 16 + iota],
                       jnp.full((16,), jnp.int32(SENTINEL)))
    nwch = (wcount + 31) >> 4

    bufs = (qb0, qb1)
    sems = (qs0, qs1)

    def fire_q(cblk, q, buf, sem):
        src = w_hbm.at[pl.ds((q % NQ) * QD, QD),
                       pl.ds(pl.multiple_of(cblk * 128, 128), 128)]
        return pltpu.async_copy(src, buf, sem)

    def drain_q(buf, sem):
        pltpu.make_async_copy(
            w_hbm.at[pl.ds(0, QD), pl.ds(0, 128)], buf, sem).wait()

    def drain_out(slot, nwords):
        pltpu.make_async_copy(
            stag.at[pl.ds(slot * D_MODEL, nwords)],
            out_hbm.at[pl.ds(0, nwords)],
            osem.at[slot]).wait()

    fire_q(c0, 0, qb0, qs0)
    fire_q(c0, 1, qb1, qs1)

    def extract_rows(buf, nm, rows_per_tok, colv_fn, out_off_fn):
        def tok(i, drain):
            ch = blist[pl.ds((i >> 4) * 16, 16)]
            e = jnp.take_along_axis(
                ch, jnp.full((16,), i & 15, jnp.int32), axis=0)
            colv = colv_fn(e)
            t = e[0] & 8191
            slot = i & (RING - 1)
            sbase = slot * D_MODEL
            if drain:
                drain_out(slot, rows_per_tok)
            for j in range(rows_per_tok // 16):
                val = plsc.load_gather(buf, [iota + 16 * j, colv])
                stag[pl.ds(sbase + 16 * j, 16)] = val
            pltpu.async_copy(
                stag.at[pl.ds(pl.multiple_of(sbase, 8), rows_per_tok)],
                out_hbm.at[pl.ds(
                    pl.multiple_of(out_off_fn(t), 8), rows_per_tok)],
                osem.at[slot])
            return 0

        lax.fori_loop(0, jnp.minimum(nm, RING),
                      lambda i, c: tok(i, False), 0)
        lax.fori_loop(RING, jnp.maximum(nm, RING),
                      lambda i, c: tok(i, True), 0)
        lax.fori_loop(0, jnp.minimum(nm, RING),
                      lambda s, c: (drain_out(s, rows_per_tok), c)[1], 0)

    def block_body(cb, carry):
        def bl_body(k, off):
            ch = wlist[pl.ds(k * 16, 16)]
            return compact(blist, off, ch, (ch >> 20) == cb)

        nm = lax.fori_loop(0, nwch, bl_body, 0)

        for q in range(NQ):
            b = q & 1
            drain_q(bufs[b], sems[b])
            extract_rows(
                bufs[b], nm, QD,
                lambda e: (e >> 13) & 127,
                lambda t, q=q: t * D_MODEL + q * QD)
            nxt = q + 2
            fire_q(jnp.minimum(cb + nxt // NQ, NBLK - 1), nxt % NQ,
                   bufs[b], sems[b])
        return carry

    lax.fori_loop(c0, c1, block_body, 0)
    drain_q(qb0, qs0)
    drain_q(qb1, qs1)

    @pl.when(wid == NW - 1)
    def _tail():
        def tscan(k, off):
            v = xtile[pl.ds(k * 16, 16)]
            e = (v << 13) | (iota + k * 16)
            return compact(blist, off, e, v >= TAIL0)

        nm = lax.fori_loop(0, XCH, tscan, 0)

        for q in range(NQ):
            pltpu.sync_copy(tail_hbm.at[pl.ds(q * QD, QD)], qb0)
            extract_rows(
                qb0, nm, QD,
                lambda e: (e >> 13) - TAILB0,
                lambda t, q=q: t * D_MODEL + q * QD)


@functools.partial(
    pl.kernel,
    out_type=jax.ShapeDtypeStruct((T * D_MODEL,), jnp.float32),
    mesh=plsc.VectorSubcoreMesh(core_axis_name="c", subcore_axis_name="s"),
    compiler_params=pltpu.CompilerParams(needs_layout_passes=False),
    scratch_types=[
        pltpu.VMEM((T,), jnp.int32),
        pltpu.VMEM((T + 32,), jnp.int32),
        pltpu.VMEM((T + 32,), jnp.int32),
        pltpu.VMEM((QD, 128), jnp.float32),
        pltpu.VMEM((QD, 128), jnp.float32),
        pltpu.VMEM((RING * D_MODEL,), jnp.float32),
        pltpu.SemaphoreType.DMA,
        pltpu.SemaphoreType.DMA,
        pltpu.SemaphoreType.DMA((RING,)),
    ],
)
def _embed_call(w_hbm, tail_hbm, x_hbm, out_hbm, xtile, wlist, blist,
                qb0, qb1, stag, qs0, qs1, osem):
    _embed_body(w_hbm, tail_hbm, x_hbm, out_hbm, xtile, wlist, blist,
                qb0, qb1, stag, qs0, qs1, osem)


def kernel(x, W_E):
    b, s = x.shape
    xf = x.reshape(-1).astype(jnp.int32)
    tail = lax.slice(W_E, (0, TAILB0), (D_MODEL, D_VOCAB))
    out = _embed_call(W_E, tail, xf)
    return out.reshape(b, s, D_MODEL)
